# baseline (speedup 1.0000x reference)
import jax
import jax.numpy as jnp
import numpy as np
from jax.experimental import pallas as pl
from jax.experimental.pallas import tpu as pltpu
from jax.experimental.pallas import tpu_sc as plsc

D_MODEL = 1024
N_HEADS = 16
DH = D_MODEL // N_HEADS
N_INPUT = 64
K_TOP = 8
B = 2
S = 2048
BM = 256
BQ = 512
NBLK = (B * S) // BM
BLK_PER_BATCH = S // BM


def _qkv_kernel(x_ref, w_ref, b_ref, out_ref):
    x = x_ref[...]
    w = w_ref[...]
    acc = jax.lax.dot_general(x, w, (((1,), (1,)), ((), ())),
                              preferred_element_type=jnp.float32)
    acc = acc + b_ref[...]
    out_ref[0, 0] = acc.reshape(BM, N_HEADS, DH)


def _attn_kernel(q_ref, k_ref, v_ref, o_ref):
    q = q_ref[0]
    k = k_ref[0]
    v = v_ref[0]
    s = jax.lax.dot_general(q, k, (((1,), (1,)), ((), ())),
                            preferred_element_type=jnp.float32)
    s = s * jnp.float32(1.0 / np.sqrt(DH))
    m = jnp.max(s, axis=1, keepdims=True)
    p = jnp.exp(s - m)
    l = jnp.sum(p, axis=1, keepdims=True)
    p = p / l
    o_ref[0] = jnp.dot(p, v, preferred_element_type=jnp.float32)


def _out_kernel(ctx_ref, wo_ref, bo_ref, wc_ref, bc_ref,
                att_ref, fin_ref, prb_ref, ssum, smax):
    i = pl.program_id(0)
    ctxb = ctx_ref[...]
    att = jax.lax.dot_general(ctxb, wo_ref[...], (((1,), (1,)), ((), ())),
                              preferred_element_type=jnp.float32)
    att = att + bo_ref[...]
    att_ref[...] = att
    aff = jax.lax.dot_general(att, wc_ref[...], (((1,), (1,)), ((), ())),
                              preferred_element_type=jnp.float32)
    aff = aff + bc_ref[...]
    psum = jnp.sum(aff, axis=0, keepdims=True)
    pmax = jnp.max(aff, axis=0, keepdims=True)
    b = i // BLK_PER_BATCH
    rows = jax.lax.broadcasted_iota(jnp.int32, (B, 1), 0)
    mask = rows == b

    @pl.when(i == 0)
    def _():
        ssum[...] = jnp.zeros((B, N_INPUT), jnp.float32)
        smax[...] = jnp.full((B, N_INPUT), -jnp.inf, jnp.float32)

    ssum[...] = ssum[...] + jnp.where(mask, psum, 0.0)
    smax[...] = jnp.maximum(smax[...], jnp.where(mask, pmax, -jnp.inf))

    @pl.when(i == NBLK - 1)
    def _():
        ss = ssum[...]
        sm = smax[...]
        inv = jnp.float32(1.0 / (S + 1e-8))
        final = 0.5 * (ss * inv) + 0.3 * sm + 0.2 * (ss * jnp.float32(1.0 / S))
        fm = jnp.max(final, axis=1, keepdims=True)
        pe = jnp.exp(final - fm)
        probs = pe / jnp.sum(pe, axis=1, keepdims=True)
        fin_ref[...] = final
        prb_ref[...] = probs


def _sc_route_kernel(fin_hbm, prb_hbm, idx_hbm, rw_hbm,
                     fin_v, prb_v, idx_v, rw_v):
    wid_ok = (jax.lax.axis_index("c") == 0) & (jax.lax.axis_index("s") == 0)

    @pl.when(wid_ok)
    def _():
        iota = jax.lax.iota(jnp.int32, 16)
        lane8 = iota < 8
        for b in range(B):
            pltpu.sync_copy(fin_hbm.at[pl.ds(N_INPUT * b, N_INPUT)], fin_v)
            pltpu.sync_copy(prb_hbm.at[pl.ds(N_INPUT * b, N_INPUT)], prb_v)
            pairs = []
            for c in range(N_INPUT // 16):
                kk = fin_v[pl.ds(16 * c, 16)]
                vv = iota + 16 * c
                pairs.append(plsc.sort_key_val(kk, vv, descending=True))

            def merge(a, bb):
                ka, va = a
                kb, vb = bb
                mk = jnp.where(lane8, ka, jax.lax.rev(kb, (0,)))
                mv = jnp.where(lane8, va, jax.lax.rev(vb, (0,)))
                return plsc.sort_key_val(mk, mv, descending=True)

            mk, mv = merge(merge(pairs[0], pairs[1]),
                           merge(pairs[2], pairs[3]))
            thr = jnp.min(jnp.where(lane8, mk, jnp.float32(jnp.inf)))
            idx_v[...] = mv
            for c in range(N_INPUT // 16):
                fc = fin_v[pl.ds(16 * c, 16)]
                pc = prb_v[pl.ds(16 * c, 16)]
                rw_v[pl.ds(16 * c, 16)] = jnp.where(
                    fc >= thr, (1.0 - pc) + pc, (0.0 - pc) + pc)
            pltpu.sync_copy(idx_v.at[pl.ds(0, K_TOP)],
                            idx_hbm.at[pl.ds(K_TOP * b, K_TOP)])
            pltpu.sync_copy(rw_v, rw_hbm.at[pl.ds(N_INPUT * b, N_INPUT)])


def _sc_route_call(final_scores, routing_probs):
    idx1, rw1 = pl.kernel(
        _sc_route_kernel,
        out_type=[
            jax.ShapeDtypeStruct((B * K_TOP,), jnp.int32),
            jax.ShapeDtypeStruct((B * N_INPUT,), jnp.float32),
        ],
        mesh=plsc.VectorSubcoreMesh(core_axis_name="c", subcore_axis_name="s"),
        compiler_params=pltpu.CompilerParams(needs_layout_passes=False),
        scratch_types=[
            pltpu.VMEM((N_INPUT,), jnp.float32),
            pltpu.VMEM((N_INPUT,), jnp.float32),
            pltpu.VMEM((16,), jnp.int32),
            pltpu.VMEM((N_INPUT,), jnp.float32),
        ],
    )(final_scores.reshape(B * N_INPUT), routing_probs.reshape(B * N_INPUT))
    return idx1.reshape(B, K_TOP), rw1.reshape(B, N_INPUT)


def kernel(x, in_proj_w, in_proj_b, out_proj_w, out_proj_b,
           content_w, content_b, k_input):
    x2d = x.reshape(B * S, D_MODEL)

    qkv3 = pl.pallas_call(
        _qkv_kernel,
        grid=(3, NBLK),
        in_specs=[
            pl.BlockSpec((BM, D_MODEL), lambda j, i: (i, 0)),
            pl.BlockSpec((D_MODEL, D_MODEL), lambda j, i: (j, 0)),
            pl.BlockSpec((1, D_MODEL), lambda j, i: (0, j)),
        ],
        out_specs=pl.BlockSpec(
            (1, 1, BM, N_HEADS, DH),
            lambda j, i: (j, i // BLK_PER_BATCH, i % BLK_PER_BATCH, 0, 0)),
        out_shape=jax.ShapeDtypeStruct((3, B, S, N_HEADS, DH), jnp.float32),
    )(x2d, in_proj_w, in_proj_b.reshape(1, 3 * D_MODEL))

    q3 = qkv3[0].transpose(0, 2, 1, 3).reshape(B * N_HEADS, S, DH)
    k3 = qkv3[1].transpose(0, 2, 1, 3).reshape(B * N_HEADS, S, DH)
    v3 = qkv3[2].transpose(0, 2, 1, 3).reshape(B * N_HEADS, S, DH)

    ctx_h = pl.pallas_call(
        _attn_kernel,
        grid=(B * N_HEADS, S // BQ),
        in_specs=[
            pl.BlockSpec((1, BQ, DH), lambda bh, qi: (bh, qi, 0)),
            pl.BlockSpec((1, S, DH), lambda bh, qi: (bh, 0, 0)),
            pl.BlockSpec((1, S, DH), lambda bh, qi: (bh, 0, 0)),
        ],
        out_specs=pl.BlockSpec((1, BQ, DH), lambda bh, qi: (bh, qi, 0)),
        out_shape=jax.ShapeDtypeStruct((B * N_HEADS, S, DH), jnp.float32),
    )(q3, k3, v3)

    ctx2d = (ctx_h.reshape(B, N_HEADS, S, DH)
             .transpose(0, 2, 1, 3).reshape(B * S, D_MODEL))

    att2d, final_scores, routing_probs = pl.pallas_call(
        _out_kernel,
        grid=(NBLK,),
        in_specs=[
            pl.BlockSpec((BM, D_MODEL), lambda i: (i, 0)),
            pl.BlockSpec((D_MODEL, D_MODEL), lambda i: (0, 0)),
            pl.BlockSpec((1, D_MODEL), lambda i: (0, 0)),
            pl.BlockSpec((N_INPUT, D_MODEL), lambda i: (0, 0)),
            pl.BlockSpec((1, N_INPUT), lambda i: (0, 0)),
        ],
        out_specs=[
            pl.BlockSpec((BM, D_MODEL), lambda i: (i, 0)),
            pl.BlockSpec((B, N_INPUT), lambda i: (0, 0)),
            pl.BlockSpec((B, N_INPUT), lambda i: (0, 0)),
        ],
        out_shape=[
            jax.ShapeDtypeStruct((B * S, D_MODEL), jnp.float32),
            jax.ShapeDtypeStruct((B, N_INPUT), jnp.float32),
            jax.ShapeDtypeStruct((B, N_INPUT), jnp.float32),
        ],
        scratch_shapes=[
            pltpu.VMEM((B, N_INPUT), jnp.float32),
            pltpu.VMEM((B, N_INPUT), jnp.float32),
        ],
    )(ctx2d, out_proj_w, out_proj_b.reshape(1, D_MODEL),
      content_w, content_b.reshape(1, N_INPUT))

    input_idx, routing_weights = _sc_route_call(final_scores, routing_probs)
    attended = att2d.reshape(B, S, D_MODEL)
    return input_idx, routing_weights, attended


# speedup vs baseline: 1.0289x; 1.0289x over previous
import jax
import jax.numpy as jnp
import numpy as np
from jax.experimental import pallas as pl
from jax.experimental.pallas import tpu as pltpu
from jax.experimental.pallas import tpu_sc as plsc

D_MODEL = 1024
N_HEADS = 16
DH = D_MODEL // N_HEADS
N_INPUT = 64
K_TOP = 8
B = 2
S = 2048
BM = 256
BQ = 1024
NBLK = (B * S) // BM
BLK_PER_BATCH = S // BM


def _qkv_kernel(x_ref, w_ref, b_ref, out_ref):
    x = x_ref[...]
    w = w_ref[...]
    acc = jax.lax.dot_general(x, w, (((1,), (1,)), ((), ())),
                              preferred_element_type=jnp.float32)
    acc = acc + b_ref[...]
    out_ref[0, 0] = acc.reshape(BM, N_HEADS, DH)


def _attn_kernel(q_ref, k_ref, v_ref, o_ref):
    q = q_ref[0]
    k = k_ref[0]
    v = v_ref[0]
    s = jax.lax.dot_general(q, k, (((1,), (1,)), ((), ())),
                            preferred_element_type=jnp.float32)
    s = s * jnp.float32(1.0 / np.sqrt(DH))
    m = jnp.max(s, axis=1, keepdims=True)
    p = jnp.exp(s - m)
    l = jnp.sum(p, axis=1, keepdims=True)
    p = p / l
    o_ref[0] = jnp.dot(p, v, preferred_element_type=jnp.float32)


def _out_kernel(ctx_ref, wo_ref, bo_ref, wc_ref, bc_ref,
                att_ref, fin_ref, prb_ref, ssum, smax):
    i = pl.program_id(0)
    ctxb = ctx_ref[...]
    att = jax.lax.dot_general(ctxb, wo_ref[...], (((1,), (1,)), ((), ())),
                              preferred_element_type=jnp.float32)
    att = att + bo_ref[...]
    att_ref[...] = att
    aff = jax.lax.dot_general(att, wc_ref[...], (((1,), (1,)), ((), ())),
                              preferred_element_type=jnp.float32)
    aff = aff + bc_ref[...]
    psum = jnp.sum(aff, axis=0, keepdims=True)
    pmax = jnp.max(aff, axis=0, keepdims=True)
    b = i // BLK_PER_BATCH
    rows = jax.lax.broadcasted_iota(jnp.int32, (B, 1), 0)
    mask = rows == b

    @pl.when(i == 0)
    def _():
        ssum[...] = jnp.zeros((B, N_INPUT), jnp.float32)
        smax[...] = jnp.full((B, N_INPUT), -jnp.inf, jnp.float32)

    ssum[...] = ssum[...] + jnp.where(mask, psum, 0.0)
    smax[...] = jnp.maximum(smax[...], jnp.where(mask, pmax, -jnp.inf))

    @pl.when(i == NBLK - 1)
    def _():
        ss = ssum[...]
        sm = smax[...]
        inv = jnp.float32(1.0 / (S + 1e-8))
        final = 0.5 * (ss * inv) + 0.3 * sm + 0.2 * (ss * jnp.float32(1.0 / S))
        fm = jnp.max(final, axis=1, keepdims=True)
        pe = jnp.exp(final - fm)
        probs = pe / jnp.sum(pe, axis=1, keepdims=True)
        fin_ref[...] = final
        prb_ref[...] = probs


def _sc_route_kernel(fin_hbm, prb_hbm, idx_hbm, rw_hbm,
                     fin_v, prb_v, idx_v, rw_v):
    wid_ok = (jax.lax.axis_index("c") == 0) & (jax.lax.axis_index("s") == 0)

    @pl.when(wid_ok)
    def _():
        iota = jax.lax.iota(jnp.int32, 16)
        lane8 = iota < 8
        for b in range(B):
            pltpu.sync_copy(fin_hbm.at[pl.ds(N_INPUT * b, N_INPUT)], fin_v)
            pltpu.sync_copy(prb_hbm.at[pl.ds(N_INPUT * b, N_INPUT)], prb_v)
            pairs = []
            for c in range(N_INPUT // 16):
                kk = fin_v[pl.ds(16 * c, 16)]
                vv = iota + 16 * c
                pairs.append(plsc.sort_key_val(kk, vv, descending=True))

            def merge(a, bb):
                ka, va = a
                kb, vb = bb
                mk = jnp.where(lane8, ka, jax.lax.rev(kb, (0,)))
                mv = jnp.where(lane8, va, jax.lax.rev(vb, (0,)))
                return plsc.sort_key_val(mk, mv, descending=True)

            mk, mv = merge(merge(pairs[0], pairs[1]),
                           merge(pairs[2], pairs[3]))
            thr = jnp.min(jnp.where(lane8, mk, jnp.float32(jnp.inf)))
            idx_v[...] = mv
            for c in range(N_INPUT // 16):
                fc = fin_v[pl.ds(16 * c, 16)]
                pc = prb_v[pl.ds(16 * c, 16)]
                rw_v[pl.ds(16 * c, 16)] = jnp.where(
                    fc >= thr, (1.0 - pc) + pc, (0.0 - pc) + pc)
            pltpu.sync_copy(idx_v.at[pl.ds(0, K_TOP)],
                            idx_hbm.at[pl.ds(K_TOP * b, K_TOP)])
            pltpu.sync_copy(rw_v, rw_hbm.at[pl.ds(N_INPUT * b, N_INPUT)])


def _sc_route_call(final_scores, routing_probs):
    idx1, rw1 = pl.kernel(
        _sc_route_kernel,
        out_type=[
            jax.ShapeDtypeStruct((B * K_TOP,), jnp.int32),
            jax.ShapeDtypeStruct((B * N_INPUT,), jnp.float32),
        ],
        mesh=plsc.VectorSubcoreMesh(core_axis_name="c", subcore_axis_name="s"),
        compiler_params=pltpu.CompilerParams(needs_layout_passes=False),
        scratch_types=[
            pltpu.VMEM((N_INPUT,), jnp.float32),
            pltpu.VMEM((N_INPUT,), jnp.float32),
            pltpu.VMEM((16,), jnp.int32),
            pltpu.VMEM((N_INPUT,), jnp.float32),
        ],
    )(final_scores.reshape(B * N_INPUT), routing_probs.reshape(B * N_INPUT))
    return idx1.reshape(B, K_TOP), rw1.reshape(B, N_INPUT)


def kernel(x, in_proj_w, in_proj_b, out_proj_w, out_proj_b,
           content_w, content_b, k_input):
    x2d = x.reshape(B * S, D_MODEL)

    qkv3 = pl.pallas_call(
        _qkv_kernel,
        grid=(3, NBLK),
        in_specs=[
            pl.BlockSpec((BM, D_MODEL), lambda j, i: (i, 0)),
            pl.BlockSpec((D_MODEL, D_MODEL), lambda j, i: (j, 0)),
            pl.BlockSpec((1, D_MODEL), lambda j, i: (0, j)),
        ],
        out_specs=pl.BlockSpec(
            (1, 1, BM, N_HEADS, DH),
            lambda j, i: (j, i // BLK_PER_BATCH, i % BLK_PER_BATCH, 0, 0)),
        out_shape=jax.ShapeDtypeStruct((3, B, S, N_HEADS, DH), jnp.float32),
    )(x2d, in_proj_w, in_proj_b.reshape(1, 3 * D_MODEL))

    q3 = qkv3[0].transpose(0, 2, 1, 3).reshape(B * N_HEADS, S, DH)
    k3 = qkv3[1].transpose(0, 2, 1, 3).reshape(B * N_HEADS, S, DH)
    v3 = qkv3[2].transpose(0, 2, 1, 3).reshape(B * N_HEADS, S, DH)

    ctx_h = pl.pallas_call(
        _attn_kernel,
        grid=(B * N_HEADS, S // BQ),
        in_specs=[
            pl.BlockSpec((1, BQ, DH), lambda bh, qi: (bh, qi, 0)),
            pl.BlockSpec((1, S, DH), lambda bh, qi: (bh, 0, 0)),
            pl.BlockSpec((1, S, DH), lambda bh, qi: (bh, 0, 0)),
        ],
        out_specs=pl.BlockSpec((1, BQ, DH), lambda bh, qi: (bh, qi, 0)),
        out_shape=jax.ShapeDtypeStruct((B * N_HEADS, S, DH), jnp.float32),
    )(q3, k3, v3)

    ctx2d = (ctx_h.reshape(B, N_HEADS, S, DH)
             .transpose(0, 2, 1, 3).reshape(B * S, D_MODEL))

    att2d, final_scores, routing_probs = pl.pallas_call(
        _out_kernel,
        grid=(NBLK,),
        in_specs=[
            pl.BlockSpec((BM, D_MODEL), lambda i: (i, 0)),
            pl.BlockSpec((D_MODEL, D_MODEL), lambda i: (0, 0)),
            pl.BlockSpec((1, D_MODEL), lambda i: (0, 0)),
            pl.BlockSpec((N_INPUT, D_MODEL), lambda i: (0, 0)),
            pl.BlockSpec((1, N_INPUT), lambda i: (0, 0)),
        ],
        out_specs=[
            pl.BlockSpec((BM, D_MODEL), lambda i: (i, 0)),
            pl.BlockSpec((B, N_INPUT), lambda i: (0, 0)),
            pl.BlockSpec((B, N_INPUT), lambda i: (0, 0)),
        ],
        out_shape=[
            jax.ShapeDtypeStruct((B * S, D_MODEL), jnp.float32),
            jax.ShapeDtypeStruct((B, N_INPUT), jnp.float32),
            jax.ShapeDtypeStruct((B, N_INPUT), jnp.float32),
        ],
        scratch_shapes=[
            pltpu.VMEM((B, N_INPUT), jnp.float32),
            pltpu.VMEM((B, N_INPUT), jnp.float32),
        ],
    )(ctx2d, out_proj_w, out_proj_b.reshape(1, D_MODEL),
      content_w, content_b.reshape(1, N_INPUT))

    input_idx, routing_weights = _sc_route_call(final_scores, routing_probs)
    attended = att2d.reshape(B, S, D_MODEL)
    return input_idx, routing_weights, attended


# speedup vs baseline: 1.0587x; 1.0289x over previous
import jax
import jax.numpy as jnp
import numpy as np
from jax.experimental import pallas as pl
from jax.experimental.pallas import tpu as pltpu
from jax.experimental.pallas import tpu_sc as plsc

D_MODEL = 1024
N_HEADS = 16
DH = D_MODEL // N_HEADS
N_INPUT = 64
K_TOP = 8
B = 2
S = 2048
BM = 512
BQ = 1024
NBLK = (B * S) // BM
BLK_PER_BATCH = S // BM


def _qkv_kernel(x_ref, w_ref, b_ref, out_ref):
    x = x_ref[...]
    w = w_ref[...]
    acc = jax.lax.dot_general(x, w, (((1,), (1,)), ((), ())),
                              preferred_element_type=jnp.float32)
    acc = acc + b_ref[...]
    out_ref[0, 0] = acc.reshape(BM, N_HEADS, DH)


def _attn_kernel(q_ref, k_ref, v_ref, o_ref):
    q = q_ref[0]
    k = k_ref[0]
    v = v_ref[0]
    s = jax.lax.dot_general(q, k, (((1,), (1,)), ((), ())),
                            preferred_element_type=jnp.float32)
    s = s * jnp.float32(1.0 / np.sqrt(DH))
    m = jnp.max(s, axis=1, keepdims=True)
    p = jnp.exp(s - m)
    l = jnp.sum(p, axis=1, keepdims=True)
    p = p / l
    o_ref[0] = jnp.dot(p, v, preferred_element_type=jnp.float32)


def _out_kernel(ctx_ref, wo_ref, bo_ref, wc_ref, bc_ref,
                att_ref, fin_ref, prb_ref, ssum, smax):
    i = pl.program_id(0)
    ctxb = ctx_ref[...]
    att = jax.lax.dot_general(ctxb, wo_ref[...], (((1,), (1,)), ((), ())),
                              preferred_element_type=jnp.float32)
    att = att + bo_ref[...]
    att_ref[...] = att
    aff = jax.lax.dot_general(att, wc_ref[...], (((1,), (1,)), ((), ())),
                              preferred_element_type=jnp.float32)
    aff = aff + bc_ref[...]
    psum = jnp.sum(aff, axis=0, keepdims=True)
    pmax = jnp.max(aff, axis=0, keepdims=True)
    b = i // BLK_PER_BATCH
    rows = jax.lax.broadcasted_iota(jnp.int32, (B, 1), 0)
    mask = rows == b

    @pl.when(i == 0)
    def _():
        ssum[...] = jnp.zeros((B, N_INPUT), jnp.float32)
        smax[...] = jnp.full((B, N_INPUT), -jnp.inf, jnp.float32)

    ssum[...] = ssum[...] + jnp.where(mask, psum, 0.0)
    smax[...] = jnp.maximum(smax[...], jnp.where(mask, pmax, -jnp.inf))

    @pl.when(i == NBLK - 1)
    def _():
        ss = ssum[...]
        sm = smax[...]
        inv = jnp.float32(1.0 / (S + 1e-8))
        final = 0.5 * (ss * inv) + 0.3 * sm + 0.2 * (ss * jnp.float32(1.0 / S))
        fm = jnp.max(final, axis=1, keepdims=True)
        pe = jnp.exp(final - fm)
        probs = pe / jnp.sum(pe, axis=1, keepdims=True)
        fin_ref[...] = final
        prb_ref[...] = probs


def _sc_route_kernel(fin_hbm, prb_hbm, idx_hbm, rw_hbm,
                     fin_v, prb_v, idx_v, rw_v):
    wid_ok = (jax.lax.axis_index("c") == 0) & (jax.lax.axis_index("s") == 0)

    @pl.when(wid_ok)
    def _():
        iota = jax.lax.iota(jnp.int32, 16)
        lane8 = iota < 8
        for b in range(B):
            pltpu.sync_copy(fin_hbm.at[pl.ds(N_INPUT * b, N_INPUT)], fin_v)
            pltpu.sync_copy(prb_hbm.at[pl.ds(N_INPUT * b, N_INPUT)], prb_v)
            pairs = []
            for c in range(N_INPUT // 16):
                kk = fin_v[pl.ds(16 * c, 16)]
                vv = iota + 16 * c
                pairs.append(plsc.sort_key_val(kk, vv, descending=True))

            def merge(a, bb):
                ka, va = a
                kb, vb = bb
                mk = jnp.where(lane8, ka, jax.lax.rev(kb, (0,)))
                mv = jnp.where(lane8, va, jax.lax.rev(vb, (0,)))
                return plsc.sort_key_val(mk, mv, descending=True)

            mk, mv = merge(merge(pairs[0], pairs[1]),
                           merge(pairs[2], pairs[3]))
            thr = jnp.min(jnp.where(lane8, mk, jnp.float32(jnp.inf)))
            idx_v[...] = mv
            for c in range(N_INPUT // 16):
                fc = fin_v[pl.ds(16 * c, 16)]
                pc = prb_v[pl.ds(16 * c, 16)]
                rw_v[pl.ds(16 * c, 16)] = jnp.where(
                    fc >= thr, (1.0 - pc) + pc, (0.0 - pc) + pc)
            pltpu.sync_copy(idx_v.at[pl.ds(0, K_TOP)],
                            idx_hbm.at[pl.ds(K_TOP * b, K_TOP)])
            pltpu.sync_copy(rw_v, rw_hbm.at[pl.ds(N_INPUT * b, N_INPUT)])


def _sc_route_call(final_scores, routing_probs):
    idx1, rw1 = pl.kernel(
        _sc_route_kernel,
        out_type=[
            jax.ShapeDtypeStruct((B * K_TOP,), jnp.int32),
            jax.ShapeDtypeStruct((B * N_INPUT,), jnp.float32),
        ],
        mesh=plsc.VectorSubcoreMesh(core_axis_name="c", subcore_axis_name="s"),
        compiler_params=pltpu.CompilerParams(needs_layout_passes=False),
        scratch_types=[
            pltpu.VMEM((N_INPUT,), jnp.float32),
            pltpu.VMEM((N_INPUT,), jnp.float32),
            pltpu.VMEM((16,), jnp.int32),
            pltpu.VMEM((N_INPUT,), jnp.float32),
        ],
    )(final_scores.reshape(B * N_INPUT), routing_probs.reshape(B * N_INPUT))
    return idx1.reshape(B, K_TOP), rw1.reshape(B, N_INPUT)


def kernel(x, in_proj_w, in_proj_b, out_proj_w, out_proj_b,
           content_w, content_b, k_input):
    x2d = x.reshape(B * S, D_MODEL)

    qkv3 = pl.pallas_call(
        _qkv_kernel,
        grid=(3, NBLK),
        in_specs=[
            pl.BlockSpec((BM, D_MODEL), lambda j, i: (i, 0)),
            pl.BlockSpec((D_MODEL, D_MODEL), lambda j, i: (j, 0)),
            pl.BlockSpec((1, D_MODEL), lambda j, i: (0, j)),
        ],
        out_specs=pl.BlockSpec(
            (1, 1, BM, N_HEADS, DH),
            lambda j, i: (j, i // BLK_PER_BATCH, i % BLK_PER_BATCH, 0, 0)),
        out_shape=jax.ShapeDtypeStruct((3, B, S, N_HEADS, DH), jnp.float32),
    )(x2d, in_proj_w, in_proj_b.reshape(1, 3 * D_MODEL))

    q3 = qkv3[0].transpose(0, 2, 1, 3).reshape(B * N_HEADS, S, DH)
    k3 = qkv3[1].transpose(0, 2, 1, 3).reshape(B * N_HEADS, S, DH)
    v3 = qkv3[2].transpose(0, 2, 1, 3).reshape(B * N_HEADS, S, DH)

    ctx_h = pl.pallas_call(
        _attn_kernel,
        grid=(B * N_HEADS, S // BQ),
        in_specs=[
            pl.BlockSpec((1, BQ, DH), lambda bh, qi: (bh, qi, 0)),
            pl.BlockSpec((1, S, DH), lambda bh, qi: (bh, 0, 0)),
            pl.BlockSpec((1, S, DH), lambda bh, qi: (bh, 0, 0)),
        ],
        out_specs=pl.BlockSpec((1, BQ, DH), lambda bh, qi: (bh, qi, 0)),
        out_shape=jax.ShapeDtypeStruct((B * N_HEADS, S, DH), jnp.float32),
    )(q3, k3, v3)

    ctx2d = (ctx_h.reshape(B, N_HEADS, S, DH)
             .transpose(0, 2, 1, 3).reshape(B * S, D_MODEL))

    att2d, final_scores, routing_probs = pl.pallas_call(
        _out_kernel,
        grid=(NBLK,),
        in_specs=[
            pl.BlockSpec((BM, D_MODEL), lambda i: (i, 0)),
            pl.BlockSpec((D_MODEL, D_MODEL), lambda i: (0, 0)),
            pl.BlockSpec((1, D_MODEL), lambda i: (0, 0)),
            pl.BlockSpec((N_INPUT, D_MODEL), lambda i: (0, 0)),
            pl.BlockSpec((1, N_INPUT), lambda i: (0, 0)),
        ],
        out_specs=[
            pl.BlockSpec((BM, D_MODEL), lambda i: (i, 0)),
            pl.BlockSpec((B, N_INPUT), lambda i: (0, 0)),
            pl.BlockSpec((B, N_INPUT), lambda i: (0, 0)),
        ],
        out_shape=[
            jax.ShapeDtypeStruct((B * S, D_MODEL), jnp.float32),
            jax.ShapeDtypeStruct((B, N_INPUT), jnp.float32),
            jax.ShapeDtypeStruct((B, N_INPUT), jnp.float32),
        ],
        scratch_shapes=[
            pltpu.VMEM((B, N_INPUT), jnp.float32),
            pltpu.VMEM((B, N_INPUT), jnp.float32),
        ],
    )(ctx2d, out_proj_w, out_proj_b.reshape(1, D_MODEL),
      content_w, content_b.reshape(1, N_INPUT))

    input_idx, routing_weights = _sc_route_call(final_scores, routing_probs)
    attended = att2d.reshape(B, S, D_MODEL)
    return input_idx, routing_weights, attended
